# baseline (speedup 1.0000x reference)
import functools

import jax
import jax.numpy as jnp
from jax import lax
from jax.experimental import pallas as pl
from jax.experimental.pallas import tpu as pltpu
from jax.experimental.pallas import tpu_sc as plsc

VOCAB = 1000000
EMBED = 64
SEQ = 20
BATCH = 16384
B_TOTAL = BATCH * SEQ

_INFO = plsc.get_sparse_core_info()
_NC = _INFO.num_cores
_NS = _INFO.num_subcores
_NW = _NC * _NS
_PER_W = B_TOTAL // _NW
_CHUNK = 32 * SEQ
_NCHUNK = _PER_W // _CHUNK


def _embed_kernel(idx_hbm, table_hbm, out_hbm, idx_v, rows, stage, gsem,
                  osem):
    wid = lax.axis_index("s") * _NC + lax.axis_index("c")
    base = wid * _PER_W
    pltpu.sync_copy(idx_hbm.at[pl.ds(base, _PER_W)], idx_v)
    lanes = lax.iota(jnp.int32, 16)

    def chunk_body(i, _):
        pltpu.async_copy(
            table_hbm.at[idx_v.at[pl.ds(i * _CHUNK, _CHUNK)]], rows,
            gsem).wait()
        bt = wid * 4 + i // 4
        c0 = (i % 4) * 32

        def transform(j, _):
            for e8 in range(8):
                for r in range(8):
                    col = jnp.full((16,), e8 * 8 + r, dtype=jnp.int32)
                    for h in range(2):
                        rowv = (lanes + h * 16) * SEQ + j
                        v = plsc.load_gather(rows, [rowv, col])
                        stage[j, e8, r, pl.ds(h * 16, 16)] = v
            return _

        lax.fori_loop(0, SEQ, transform, None)

        def issue(j, _):
            for e8 in range(8):
                pltpu.async_copy(
                    stage.at[j, e8],
                    out_hbm.at[j, e8, bt, :, pl.ds(c0, 32)], osem)
            return _

        def drain(j, _):
            for e8 in range(8):
                pltpu.make_async_copy(
                    stage.at[j, e8],
                    out_hbm.at[j, e8, bt, :, pl.ds(c0, 32)], osem).wait()
            return _

        lax.fori_loop(0, SEQ, issue, None)
        lax.fori_loop(0, SEQ, drain, None)
        return _

    lax.fori_loop(0, _NCHUNK, chunk_body, None)


def _sc_gather(idx_flat, table):
    mesh = plsc.VectorSubcoreMesh(core_axis_name="c", subcore_axis_name="s")
    k = functools.partial(
        pl.kernel,
        mesh=mesh,
        out_type=jax.ShapeDtypeStruct((SEQ, 8, 128, 8, 128), jnp.float32),
        scratch_types=[
            pltpu.VMEM((_PER_W,), jnp.int32),
            pltpu.VMEM((_CHUNK, EMBED), jnp.float32),
            pltpu.VMEM((SEQ, 8, 8, 32), jnp.float32),
            pltpu.SemaphoreType.DMA,
            pltpu.SemaphoreType.DMA,
        ],
        compiler_params=pltpu.CompilerParams(
            use_tc_tiling_on_sc=False, needs_layout_passes=False),
    )(_embed_kernel)
    return k(idx_flat, table)


def kernel(input_ids, weight):
    idx_flat = input_ids.reshape(-1).astype(jnp.int32)
    out5 = _sc_gather(idx_flat, weight)
    return out5.transpose(2, 4, 0, 1, 3).reshape(BATCH, SEQ, EMBED)


# speedup vs baseline: 1.3111x; 1.3111x over previous
import functools

import jax
import jax.numpy as jnp
from jax import lax
from jax.experimental import pallas as pl
from jax.experimental.pallas import tpu as pltpu
from jax.experimental.pallas import tpu_sc as plsc

VOCAB = 1000000
EMBED = 64
B_TOTAL = 16384 * 20

_INFO = plsc.get_sparse_core_info()
_NC = _INFO.num_cores
_NS = _INFO.num_subcores
_NW = _NC * _NS
_PER_W = B_TOTAL // _NW
_CHUNK = 640
_NCHUNK = _PER_W // _CHUNK


def _embed_kernel(idx_hbm, table_hbm, out_hbm, idx_v, rows0, rows1, gsem0,
                  gsem1, osem0, osem1):
    wid = lax.axis_index("s") * _NC + lax.axis_index("c")
    base = wid * _PER_W
    pltpu.sync_copy(idx_hbm.at[pl.ds(base, _PER_W)], idx_v)
    rows = (rows0, rows1)
    gsem = (gsem0, gsem1)
    osem = (osem0, osem1)

    def gather(i):
        s = i % 2
        return pltpu.async_copy(
            table_hbm.at[idx_v.at[pl.ds(i * _CHUNK, _CHUNK)]], rows[s], gsem[s])

    def store(i):
        s = i % 2
        return pltpu.async_copy(
            rows[s], out_hbm.at[pl.ds(base + i * _CHUNK, _CHUNK)], osem[s])

    stores = [None, None]
    gather(0)
    for i in range(_NCHUNK):
        s = i % 2
        pltpu.make_async_copy(
            table_hbm.at[idx_v.at[pl.ds(i * _CHUNK, _CHUNK)]], rows[s],
            gsem[s]).wait()
        if i + 1 < _NCHUNK:
            if stores[(i + 1) % 2] is not None:
                stores[(i + 1) % 2].wait()
            gather(i + 1)
        stores[s] = store(i)
    stores[0].wait()
    stores[1].wait()


def _sc_gather(idx_flat, table):
    mesh = plsc.VectorSubcoreMesh(core_axis_name="c", subcore_axis_name="s")
    k = functools.partial(
        pl.kernel,
        mesh=mesh,
        out_type=jax.ShapeDtypeStruct((B_TOTAL, EMBED), jnp.float32),
        scratch_types=[
            pltpu.VMEM((_PER_W,), jnp.int32),
            pltpu.VMEM((_CHUNK, EMBED), jnp.float32),
            pltpu.VMEM((_CHUNK, EMBED), jnp.float32),
            pltpu.SemaphoreType.DMA,
            pltpu.SemaphoreType.DMA,
            pltpu.SemaphoreType.DMA,
            pltpu.SemaphoreType.DMA,
        ],
        compiler_params=pltpu.CompilerParams(use_tc_tiling_on_sc=False),
    )(_embed_kernel)
    return k(idx_flat, table)


def kernel(input_ids, weight):
    idx_flat = input_ids.reshape(-1).astype(jnp.int32)
    out = _sc_gather(idx_flat, weight)
    return out.reshape(input_ids.shape + (EMBED,))
